# baseline (speedup 1.0000x reference)
import functools

import jax
import jax.numpy as jnp
from jax import lax
from jax.experimental import pallas as pl
from jax.experimental.pallas import tpu as pltpu
from jax.experimental.pallas import tpu_sc as plsc

D_MODEL = 64
SCALE = 8.0
CG = 128


@functools.lru_cache(maxsize=None)
def _make_kernel(n_total):
    info = plsc.get_sparse_core_info()
    nc, ns, lanes = info.num_cores, info.num_subcores, info.num_lanes
    nw = nc * ns
    assert n_total % (nw * CG) == 0
    n_per_w = n_total // nw
    n_chunks = n_per_w // CG
    vregs_per_row = D_MODEL // lanes

    mesh = plsc.VectorSubcoreMesh(core_axis_name="c", subcore_axis_name="s")

    @functools.partial(
        pl.kernel,
        mesh=mesh,
        out_type=jax.ShapeDtypeStruct((n_total, D_MODEL), jnp.float32),
        scratch_types=[
            pltpu.VMEM((n_per_w,), jnp.int32),
            pltpu.VMEM((CG, D_MODEL), jnp.float32),
            pltpu.SemaphoreType.DMA,
        ],
        compiler_params=pltpu.CompilerParams(use_tc_tiling_on_sc=False),
    )
    def emb_kernel(table_hbm, idx_hbm, out_hbm, idx_v, rows_v, sem):
        wid = lax.axis_index("s") * nc + lax.axis_index("c")
        base = wid * n_per_w
        pltpu.sync_copy(idx_hbm.at[pl.ds(base, n_per_w)], idx_v)

        def chunk_body(j, carry):
            off = j * CG
            pltpu.async_copy(
                table_hbm.at[idx_v.at[pl.ds(off, CG)]], rows_v, sem
            ).wait()

            def row_body(i, c2):
                for c in range(vregs_per_row):
                    sl = pl.ds(c * lanes, lanes)
                    rows_v[i, sl] = rows_v[i, sl] * SCALE
                return c2

            lax.fori_loop(0, CG, row_body, 0)
            pltpu.sync_copy(rows_v, out_hbm.at[pl.ds(base + off, CG)])
            return carry

        lax.fori_loop(0, n_chunks, chunk_body, 0)

    return emb_kernel


def kernel(x, table):
    b, l = x.shape
    n_total = b * l
    idx = x.reshape(n_total).astype(jnp.int32)
    out = _make_kernel(n_total)(table, idx)
    return out.reshape(b, l, D_MODEL)


# speedup vs baseline: 1.2061x; 1.2061x over previous
import functools

import jax
import jax.numpy as jnp
from jax import lax
from jax.experimental import pallas as pl
from jax.experimental.pallas import tpu as pltpu
from jax.experimental.pallas import tpu_sc as plsc

D_MODEL = 64
SCALE = 8.0
CG = 128
NB = 4
R_UNROLL = 4


@functools.lru_cache(maxsize=None)
def _make_kernel(n_total):
    info = plsc.get_sparse_core_info()
    nc, ns, lanes = info.num_cores, info.num_subcores, info.num_lanes
    nw = nc * ns
    assert n_total % (nw * CG * NB) == 0
    n_per_w = n_total // nw
    n_chunks = n_per_w // CG
    n_groups = n_chunks // NB
    vregs_per_row = D_MODEL // lanes

    mesh = plsc.VectorSubcoreMesh(core_axis_name="c", subcore_axis_name="s")

    @functools.partial(
        pl.kernel,
        mesh=mesh,
        out_type=jax.ShapeDtypeStruct((n_total, D_MODEL), jnp.float32),
        scratch_types=[
            pltpu.VMEM((n_per_w,), jnp.int32),
            pltpu.VMEM((NB, CG, D_MODEL), jnp.float32),
        ]
        + [pltpu.SemaphoreType.DMA] * (2 * NB),
        compiler_params=pltpu.CompilerParams(use_tc_tiling_on_sc=False),
    )
    def emb_kernel(table_hbm, idx_hbm, out_hbm, idx_v, rows, *sems):
        gsems, ssems = sems[:NB], sems[NB:]
        wid = lax.axis_index("s") * nc + lax.axis_index("c")
        base = wid * n_per_w
        pltpu.sync_copy(idx_hbm.at[pl.ds(base, n_per_w)], idx_v)

        def gather_desc(chunk, b):
            return pltpu.make_async_copy(
                table_hbm.at[idx_v.at[pl.ds(chunk * CG, CG)]],
                rows.at[b],
                gsems[b],
            )

        def store_desc(chunk, b):
            return pltpu.make_async_copy(
                rows.at[b],
                out_hbm.at[pl.ds(base + chunk * CG, CG)],
                ssems[b],
            )

        def scale_buf(b):
            def body(k, c):
                for r in range(R_UNROLL):
                    i = k * R_UNROLL + r
                    for v in range(vregs_per_row):
                        sl = pl.ds(v * lanes, lanes)
                        rows[b, i, sl] = rows[b, i, sl] * SCALE
                return c

            lax.fori_loop(0, CG // R_UNROLL, body, 0)

        for b in range(NB - 1):
            gather_desc(b, b).start()

        def group(g, carry):
            for b in range(NB):
                j = g * NB + b
                bf = (b + NB - 1) % NB
                jf = j + NB - 1
                gather_desc(j, b).wait()
                scale_buf(b)
                if b == 0:
                    @pl.when(g > 0)
                    def _():
                        store_desc(j - 1, bf).wait()
                else:
                    store_desc(j - 1, bf).wait()
                @pl.when(jf < n_chunks)
                def _():
                    gather_desc(jf, bf).start()
                store_desc(j, b).start()
            return carry

        lax.fori_loop(0, n_groups, group, 0)
        store_desc(n_chunks - 1, NB - 1).wait()

    return emb_kernel


def kernel(x, table):
    b, l = x.shape
    n_total = b * l
    idx = x.reshape(n_total).astype(jnp.int32)
    out = _make_kernel(n_total)(table, idx)
    return out.reshape(b, l, D_MODEL)
